# Initial kernel scaffold; baseline (speedup 1.0000x reference)
import jax
import jax.numpy as jnp
from jax.experimental import pallas as pl


def kernel(x, edge_index, edge_weight, W_l, b_l, W_r):
    raise NotImplementedError("write your pallas kernel here")


# speedup vs baseline: 5.1365x; 5.1365x over previous
import functools

import jax
import jax.numpy as jnp
from jax import lax
from jax.experimental import pallas as pl
from jax.experimental.pallas import tpu as pltpu
from jax.experimental.pallas import tpu_sc as plsc

_NC = 2
_NS = 16
_NW = _NC * _NS
_L = 16


def _make_agg(n, d, nb, k):
  rows_per_tile = -(-n // (_NS * k)) * k
  n_pad = rows_per_tile * _NS
  nz = rows_per_tile // k
  mesh = plsc.VectorSubcoreMesh(core_axis_name="c", subcore_axis_name="s")

  @functools.partial(
      pl.kernel,
      out_type=jax.ShapeDtypeStruct((_NC, n_pad, d), jnp.float32),
      mesh=mesh,
      scratch_types=[
          pltpu.VMEM((nb, k), jnp.int32),
          pltpu.VMEM((nb, k), jnp.int32),
          pltpu.VMEM((nb, k), jnp.float32),
          pltpu.VMEM((k, d), jnp.float32),
          pltpu.VMEM_SHARED((n_pad, d), jnp.float32),
          pltpu.SemaphoreType.DMA,
      ],
  )
  def agg(x_hbm, row_hbm, col_hbm, w_hbm, out_hbm,
          colv, rowv, wv, rows, acc, sem):
    c = lax.axis_index("c")
    s = lax.axis_index("s")
    wid = c * _NS + s

    pltpu.sync_copy(col_hbm.at[wid], colv)
    pltpu.sync_copy(row_hbm.at[wid], rowv)
    pltpu.sync_copy(w_hbm.at[wid], wv)

    zero = jnp.zeros((_L,), jnp.float32)

    def zstore(i, _):
      r = i // (d // _L)
      col0 = (i % (d // _L)) * _L
      rows[r, pl.ds(col0, _L)] = zero
      return 0

    lax.fori_loop(0, k * (d // _L), zstore, 0)
    for t in range(nz):
      pltpu.sync_copy(rows, acc.at[pl.ds(s * rows_per_tile + t * k, k)])
    plsc.subcore_barrier()

    def body(b, _):
      pltpu.async_copy(x_hbm.at[colv.at[b]], rows, sem).wait()

      def scale16(g, _):
        wvec = wv[b, pl.ds(g * _L, _L)]
        for j16 in range(_L):
          w = wvec[j16]
          j = g * _L + j16
          for t in range(d // _L):
            rows[j, pl.ds(t * _L, _L)] = rows[j, pl.ds(t * _L, _L)] * w
        return 0

      lax.fori_loop(0, k // _L, scale16, 0)

      pltpu.sync_copy(rows, acc.at[rowv.at[b]], add=True)
      return 0

    lax.fori_loop(0, nb, body, 0)

    plsc.subcore_barrier()
    pltpu.sync_copy(acc.at[pl.ds(s * rows_per_tile, rows_per_tile)],
                    out_hbm.at[c, pl.ds(s * rows_per_tile, rows_per_tile)])

  return agg


def _dense(p, x, W_l, b8, W_r):
  n, d = x.shape
  bn = 2000
  dn = (((1,), (1,)), ((), ()))

  def body(p_ref, x_ref, wl_ref, b_ref, wr_ref, o_ref):
    agg = p_ref[0] + p_ref[1]
    o_ref[...] = (
        lax.dot_general(agg, wl_ref[...], dn,
                        preferred_element_type=jnp.float32,
                        precision=lax.Precision.HIGHEST)
        + lax.dot_general(x_ref[...], wr_ref[...], dn,
                          preferred_element_type=jnp.float32,
                          precision=lax.Precision.HIGHEST)
        + b_ref[0:1, :])

  return pl.pallas_call(
      body,
      grid=(n // bn,),
      in_specs=[
          pl.BlockSpec((2, bn, d), lambda i: (0, i, 0)),
          pl.BlockSpec((bn, d), lambda i: (i, 0)),
          pl.BlockSpec((d, d), lambda i: (0, 0)),
          pl.BlockSpec((8, d), lambda i: (0, 0)),
          pl.BlockSpec((d, d), lambda i: (0, 0)),
      ],
      out_specs=pl.BlockSpec((bn, d), lambda i: (i, 0)),
      out_shape=jax.ShapeDtypeStruct((n, d), jnp.float32),
  )(p, x, W_l, b8, W_r)


_K = 128


def kernel(x, edge_index, edge_weight, W_l, b_l, W_r):
  n, d = x.shape
  e = edge_weight.shape[0]
  nb = -(-e // (_NW * _K))
  pad = _NW * nb * _K - e
  row = jnp.pad(edge_index[0].astype(jnp.int32), (0, pad)).reshape(_NW, nb, _K)
  col = jnp.pad(edge_index[1].astype(jnp.int32), (0, pad)).reshape(_NW, nb, _K)
  w2 = jnp.pad(edge_weight, (0, pad)).reshape(_NW, nb, _K)
  p = _make_agg(n, d, nb, _K)(x, row, col, w2)
  b8 = jnp.broadcast_to(b_l.reshape(1, d), (8, d))
  return _dense(p, x, W_l, b8, W_r)
